# Initial kernel scaffold; baseline (speedup 1.0000x reference)
import jax
import jax.numpy as jnp
from jax.experimental import pallas as pl


def kernel(x, table):
    raise NotImplementedError("write your pallas kernel here")


# speedup vs baseline: 6.4752x; 6.4752x over previous
import functools

import jax
import jax.numpy as jnp
from jax import lax
from jax.experimental import pallas as pl
from jax.experimental.pallas import tpu as pltpu
from jax.experimental.pallas import tpu_sc as plsc

D_MODEL = 128


@functools.lru_cache(maxsize=None)
def _make_gather(B, V, D):
    info = plsc.get_sparse_core_info()
    NC, NS = info.num_cores, info.num_subcores
    NW = NC * NS
    assert B % NW == 0
    b_per_w = B // NW
    CH = 800
    assert b_per_w % CH == 0
    n_ch = b_per_w // CH
    mesh = plsc.VectorSubcoreMesh(core_axis_name="c", subcore_axis_name="s")

    @functools.partial(
        pl.kernel,
        mesh=mesh,
        out_type=jax.ShapeDtypeStruct((B, D), jnp.float32),
        scratch_types=[
            pltpu.VMEM((CH,), jnp.int32),
            pltpu.VMEM((CH, D), jnp.float32),
            pltpu.SemaphoreType.DMA,
        ],
    )
    def k(idx_hbm, table_hbm, out_hbm, idx_v, rows_v, sem):
        wid = lax.axis_index("s") * NC + lax.axis_index("c")
        base = wid * b_per_w

        def body(i, carry):
            off = base + i * CH
            pltpu.sync_copy(idx_hbm.at[pl.ds(off, CH)], idx_v)
            pltpu.async_copy(table_hbm.at[idx_v], rows_v, sem).wait()
            pltpu.sync_copy(rows_v, out_hbm.at[pl.ds(off, CH)])
            return carry

        lax.fori_loop(0, n_ch, body, 0)

    return k


def kernel(x, table):
    b, t = x.shape
    flat = x.reshape(b * t).astype(jnp.int32)
    out = _make_gather(b * t, table.shape[0], table.shape[1])(flat, table)
    return out.reshape(b, t, table.shape[1])


# speedup vs baseline: 6.5165x; 1.0064x over previous
import functools

import jax
import jax.numpy as jnp
from jax import lax
from jax.experimental import pallas as pl
from jax.experimental.pallas import tpu as pltpu
from jax.experimental.pallas import tpu_sc as plsc


@functools.lru_cache(maxsize=None)
def _make_gather(B, V, D):
    info = plsc.get_sparse_core_info()
    NC, NS = info.num_cores, info.num_subcores
    NW = NC * NS
    assert B % NW == 0
    b_per_w = B // NW
    CH = 400
    assert b_per_w % (2 * CH) == 0
    n_half = b_per_w // (2 * CH)
    mesh = plsc.VectorSubcoreMesh(core_axis_name="c", subcore_axis_name="s")

    @functools.partial(
        pl.kernel,
        mesh=mesh,
        out_type=jax.ShapeDtypeStruct((B, D), jnp.float32),
        scratch_types=[
            pltpu.VMEM((b_per_w,), jnp.int32),
            pltpu.VMEM((2, CH, D), jnp.float32),
            pltpu.SemaphoreType.DMA,
            pltpu.SemaphoreType.DMA,
        ],
    )
    def k(idx_hbm, table_hbm, out_hbm, idx_v, rows_v, sem0, sem1):
        wid = lax.axis_index("s") * NC + lax.axis_index("c")
        base = wid * b_per_w
        pltpu.sync_copy(idx_hbm.at[pl.ds(base, b_per_w)], idx_v)

        def gather(c, buf, sem):
            return pltpu.make_async_copy(
                table_hbm.at[idx_v.at[pl.ds(c * CH, CH)]], rows_v.at[buf], sem
            )

        gather(0, 0, sem0).start()

        def body(g, carry):
            c = 2 * g
            gather(c + 1, 1, sem1).start()
            gather(c, 0, sem0).wait()
            pltpu.sync_copy(rows_v.at[0], out_hbm.at[pl.ds(base + c * CH, CH)])
            @pl.when(g < n_half - 1)
            def _():
                gather(c + 2, 0, sem0).start()

            gather(c + 1, 1, sem1).wait()
            pltpu.sync_copy(
                rows_v.at[1], out_hbm.at[pl.ds(base + (c + 1) * CH, CH)]
            )
            return carry

        lax.fori_loop(0, n_half, body, 0)

    return k


def kernel(x, table):
    b, t = x.shape
    flat = x.reshape(b * t).astype(jnp.int32)
    out = _make_gather(b * t, table.shape[0], table.shape[1])(flat, table)
    return out.reshape(b, t, table.shape[1])


# speedup vs baseline: 14.3747x; 2.2059x over previous
import functools

import jax
import jax.numpy as jnp
from jax import lax
from jax.experimental import pallas as pl
from jax.experimental.pallas import tpu as pltpu
from jax.experimental.pallas import tpu_sc as plsc


@functools.lru_cache(maxsize=None)
def _make_gather(B, V, D):
    info = plsc.get_sparse_core_info()
    NC, NS = info.num_cores, info.num_subcores
    NW = NC * NS
    assert B % NW == 0
    b_per_w = B // NW
    CH = 400
    assert b_per_w % (2 * CH) == 0
    n_half = b_per_w // (2 * CH)
    mesh = plsc.VectorSubcoreMesh(core_axis_name="c", subcore_axis_name="s")

    @functools.partial(
        pl.kernel,
        mesh=mesh,
        out_type=jax.ShapeDtypeStruct((B, D), jnp.float32),
        scratch_types=[
            pltpu.VMEM((CH,), jnp.int32),
            pltpu.VMEM((CH,), jnp.int32),
            pltpu.VMEM((2, CH, D), jnp.float32),
            pltpu.VMEM_SHARED((V, D), jnp.float32),
            pltpu.SemaphoreType.DMA,
            pltpu.SemaphoreType.DMA,
        ],
    )
    def k(idx_hbm, table_hbm, out_hbm, idx0, idx1, rows_v, table_sp, sem0, sem1):
        wid = lax.axis_index("s") * NC + lax.axis_index("c")
        base = wid * b_per_w

        @pl.when(lax.axis_index("s") == 0)
        def _():
            pltpu.sync_copy(table_hbm, table_sp)

        plsc.subcore_barrier()

        idx_bufs = (idx0, idx1)

        def load_idx(c, buf):
            pltpu.sync_copy(idx_hbm.at[pl.ds(base + c * CH, CH)], idx_bufs[buf])

        def gather(buf, sem):
            return pltpu.make_async_copy(
                table_sp.at[idx_bufs[buf]], rows_v.at[buf], sem
            )

        load_idx(0, 0)
        gather(0, sem0).start()

        def body(g, carry):
            c = 2 * g
            load_idx(c + 1, 1)
            gather(1, sem1).start()
            gather(0, sem0).wait()
            pltpu.sync_copy(rows_v.at[0], out_hbm.at[pl.ds(base + c * CH, CH)])
            @pl.when(g < n_half - 1)
            def _():
                load_idx(c + 2, 0)
                gather(0, sem0).start()

            gather(1, sem1).wait()
            pltpu.sync_copy(
                rows_v.at[1], out_hbm.at[pl.ds(base + (c + 1) * CH, CH)]
            )
            return carry

        lax.fori_loop(0, n_half, body, 0)

    return k


def kernel(x, table):
    b, t = x.shape
    flat = x.reshape(b * t).astype(jnp.int32)
    out = _make_gather(b * t, table.shape[0], table.shape[1])(flat, table)
    return out.reshape(b, t, table.shape[1])


# speedup vs baseline: 15.5152x; 1.0793x over previous
import functools

import jax
import jax.numpy as jnp
from jax import lax
from jax.experimental import pallas as pl
from jax.experimental.pallas import tpu as pltpu
from jax.experimental.pallas import tpu_sc as plsc


@functools.lru_cache(maxsize=None)
def _make_gather(B, V, D):
    info = plsc.get_sparse_core_info()
    NC, NS = info.num_cores, info.num_subcores
    NW = NC * NS
    assert B % NW == 0
    b_per_w = B // NW
    CH = 320
    assert b_per_w % (2 * CH) == 0
    n_half = b_per_w // (2 * CH)
    mesh = plsc.VectorSubcoreMesh(core_axis_name="c", subcore_axis_name="s")

    @functools.partial(
        pl.kernel,
        mesh=mesh,
        out_type=jax.ShapeDtypeStruct((B, D), jnp.float32),
        scratch_types=[
            pltpu.VMEM((b_per_w,), jnp.int32),
            pltpu.VMEM((2, CH, D), jnp.float32),
            pltpu.VMEM_SHARED((V, D), jnp.float32),
            pltpu.SemaphoreType.DMA,
            pltpu.SemaphoreType.DMA,
        ],
    )
    def k(idx_hbm, table_hbm, out_hbm, idx_v, rows_v, table_sp, sem0, sem1):
        wid = lax.axis_index("s") * NC + lax.axis_index("c")
        base = wid * b_per_w

        @pl.when(lax.axis_index("s") == 0)
        def _():
            pltpu.sync_copy(table_hbm, table_sp)

        pltpu.sync_copy(idx_hbm.at[pl.ds(base, b_per_w)], idx_v)
        plsc.subcore_barrier()

        def gather(c, buf, sem):
            return pltpu.make_async_copy(
                table_sp.at[idx_v.at[pl.ds(c * CH, CH)]], rows_v.at[buf], sem
            )

        gather(0, 0, sem0).start()

        def body(g, carry):
            c = 2 * g
            gather(c + 1, 1, sem1).start()
            gather(c, 0, sem0).wait()
            pltpu.sync_copy(rows_v.at[0], out_hbm.at[pl.ds(base + c * CH, CH)])
            @pl.when(g < n_half - 1)
            def _():
                gather(c + 2, 0, sem0).start()

            gather(c + 1, 1, sem1).wait()
            pltpu.sync_copy(
                rows_v.at[1], out_hbm.at[pl.ds(base + (c + 1) * CH, CH)]
            )
            return carry

        lax.fori_loop(0, n_half, body, 0)

    return k


def kernel(x, table):
    b, t = x.shape
    flat = x.reshape(b * t).astype(jnp.int32)
    out = _make_gather(b * t, table.shape[0], table.shape[1])(flat, table)
    return out.reshape(b, t, table.shape[1])


# speedup vs baseline: 15.7119x; 1.0127x over previous
import functools

import jax
import jax.numpy as jnp
from jax import lax
from jax.experimental import pallas as pl
from jax.experimental.pallas import tpu as pltpu
from jax.experimental.pallas import tpu_sc as plsc

NBUF = 4


@functools.lru_cache(maxsize=None)
def _make_gather(B, V, D):
    info = plsc.get_sparse_core_info()
    NC, NS = info.num_cores, info.num_subcores
    NW = NC * NS
    assert B % NW == 0
    b_per_w = B // NW
    CH = 160
    assert b_per_w % (NBUF * CH) == 0
    n_ch = b_per_w // CH
    n_grp = n_ch // NBUF
    mesh = plsc.VectorSubcoreMesh(core_axis_name="c", subcore_axis_name="s")

    @functools.partial(
        pl.kernel,
        mesh=mesh,
        out_type=jax.ShapeDtypeStruct((B, D), jnp.float32),
        scratch_types=[
            pltpu.VMEM((b_per_w,), jnp.int32),
            pltpu.VMEM((NBUF, CH, D), jnp.float32),
            pltpu.VMEM_SHARED((V, D), jnp.float32),
            [pltpu.SemaphoreType.DMA] * NBUF,
            [pltpu.SemaphoreType.DMA] * NBUF,
        ],
    )
    def k(idx_hbm, table_hbm, out_hbm, idx_v, rows_v, table_sp, gsem, wsem):
        wid = lax.axis_index("s") * NC + lax.axis_index("c")
        base = wid * b_per_w

        @pl.when(lax.axis_index("s") == 0)
        def _():
            pltpu.sync_copy(table_hbm, table_sp)

        pltpu.sync_copy(idx_hbm.at[pl.ds(base, b_per_w)], idx_v)
        plsc.subcore_barrier()

        def gather(c, j):
            return pltpu.make_async_copy(
                table_sp.at[idx_v.at[pl.ds(c * CH, CH)]], rows_v.at[j], gsem[j]
            )

        def write(c, j):
            return pltpu.make_async_copy(
                rows_v.at[j], out_hbm.at[pl.ds(base + c * CH, CH)], wsem[j]
            )

        gather(0, 0).start()
        gather(1, 1).start()

        def body(g, carry):
            c0 = NBUF * g
            for j in range(NBUF):
                c = c0 + j
                gather(c, j).wait()
                write(c, j).start()
                nxt = (j + 2) % NBUF

                @pl.when(c + 2 < n_ch)
                def _(c=c, nxt=nxt):
                    @pl.when(c >= 2)
                    def _():
                        write(c - 2, nxt).wait()

                    gather(c + 2, nxt).start()

            return carry

        lax.fori_loop(0, n_grp, body, 0)
        for j in range(NBUF):
            write(n_ch - NBUF + j, j).wait()

    return k


def kernel(x, table):
    b, t = x.shape
    flat = x.reshape(b * t).astype(jnp.int32)
    out = _make_gather(b * t, table.shape[0], table.shape[1])(flat, table)
    return out.reshape(b, t, table.shape[1])


# speedup vs baseline: 15.9033x; 1.0122x over previous
import functools

import jax
import jax.numpy as jnp
from jax import lax
from jax.experimental import pallas as pl
from jax.experimental.pallas import tpu as pltpu
from jax.experimental.pallas import tpu_sc as plsc

NBUF = 8


@functools.lru_cache(maxsize=None)
def _make_gather(B, V, D):
    info = plsc.get_sparse_core_info()
    NC, NS = info.num_cores, info.num_subcores
    NW = NC * NS
    assert B % NW == 0
    b_per_w = B // NW
    CH = 80
    assert b_per_w % (NBUF * CH) == 0
    n_ch = b_per_w // CH
    n_grp = n_ch // NBUF
    mesh = plsc.VectorSubcoreMesh(core_axis_name="c", subcore_axis_name="s")

    @functools.partial(
        pl.kernel,
        mesh=mesh,
        out_type=jax.ShapeDtypeStruct((B, D), jnp.float32),
        scratch_types=[
            pltpu.VMEM((b_per_w,), jnp.int32),
            pltpu.VMEM((NBUF, CH, D), jnp.float32),
            pltpu.VMEM_SHARED((V, D), jnp.float32),
            [pltpu.SemaphoreType.DMA] * NBUF,
            [pltpu.SemaphoreType.DMA] * NBUF,
        ],
    )
    def k(idx_hbm, table_hbm, out_hbm, idx_v, rows_v, table_sp, gsem, wsem):
        wid = lax.axis_index("s") * NC + lax.axis_index("c")
        base = wid * b_per_w

        @pl.when(lax.axis_index("s") == 0)
        def _():
            pltpu.sync_copy(table_hbm, table_sp)

        pltpu.sync_copy(idx_hbm.at[pl.ds(base, b_per_w)], idx_v)
        plsc.subcore_barrier()

        def gather(c, j):
            return pltpu.make_async_copy(
                table_sp.at[idx_v.at[pl.ds(c * CH, CH)]], rows_v.at[j], gsem[j]
            )

        def write(c, j):
            return pltpu.make_async_copy(
                rows_v.at[j], out_hbm.at[pl.ds(base + c * CH, CH)], wsem[j]
            )

        gather(0, 0).start()
        gather(1, 1).start()

        def body(g, carry):
            c0 = NBUF * g
            for j in range(NBUF):
                c = c0 + j
                gather(c, j).wait()
                write(c, j).start()
                nxt = (j + 2) % NBUF

                @pl.when(c + 2 < n_ch)
                def _(c=c, nxt=nxt):
                    @pl.when(c >= NBUF - 2)
                    def _():
                        write(c + 2 - NBUF, nxt).wait()

                    gather(c + 2, nxt).start()

            return carry

        lax.fori_loop(0, n_grp, body, 0)
        for j in range(NBUF):
            write(n_ch - NBUF + j, j).wait()

    return k


def kernel(x, table):
    b, t = x.shape
    flat = x.reshape(b * t).astype(jnp.int32)
    out = _make_gather(b * t, table.shape[0], table.shape[1])(flat, table)
    return out.reshape(b, t, table.shape[1])
